# baseline (device time: 484492 ns/iter reference)
import jax
import jax.numpy as jnp
from jax import lax
from jax.experimental import pallas as pl
from jax.experimental.pallas import tpu as pltpu

N_DEV = 32
BR = 64
D = 1024
H = 2048
B = N_DEV * BR
GROUPS = 4
GR = N_DEV // GROUPS
GROWS = GR * BR


def kernel(x, Win0, Wout0, Win1, Wout1, Win2, Wout2):
    xb = x.astype(jnp.bfloat16)
    wi = [w.astype(jnp.bfloat16) for w in (Win0, Win1, Win2)]
    wo = [w.astype(jnp.bfloat16) for w in (Wout0, Wout1, Wout2)]

    def body(x_ref, wi0, wi1, wi2, wo0, wo1, wo2, out_ref,
             xfull, h_ref, part_ref, rs_buf,
             sA_s, sA_r, sRS_s, sRS_r, sAG_s, sAG_r):
        me = lax.axis_index("i")
        g_me = lax.div(me, GR)

        def peer(d):
            return lax.rem(me + d, N_DEV)

        rows = lambda i: (pl.ds(i * BR, BR), slice(None))

        barrier_sem = pltpu.get_barrier_semaphore()

        def bar(d, c):
            pl.semaphore_signal(
                barrier_sem, inc=1,
                device_id=(peer(d),),
                device_id_type=pl.DeviceIdType.MESH,
            )
            return c
        lax.fori_loop(1, N_DEV, bar, 0)
        pl.semaphore_wait(barrier_sem, N_DEV - 1)

        def bcast_start(src, dst, ssem, rsem):
            def go(d, c):
                p = peer(d)
                pltpu.make_async_remote_copy(
                    src_ref=src,
                    dst_ref=dst,
                    send_sem=ssem.at[p],
                    recv_sem=rsem.at[me],
                    device_id=(p,),
                    device_id_type=pl.DeviceIdType.MESH,
                ).start()
                return c
            lax.fori_loop(1, N_DEV, go, 0)

        def bcast_wait_send(src, dst, ssem, rsem):
            def go(d, c):
                p = peer(d)
                pltpu.make_async_remote_copy(
                    src_ref=src,
                    dst_ref=dst,
                    send_sem=ssem.at[p],
                    recv_sem=rsem.at[me],
                    device_id=(p,),
                    device_id_type=pl.DeviceIdType.MESH,
                ).wait_send()
                return c
            lax.fori_loop(1, N_DEV, go, 0)

        xfull[rows(me)] = x_ref[...]
        bcast_start(x_ref, xfull.at[rows(me)], sA_s, sA_r)

        for k, (win, wout) in enumerate(((wi0, wo0), (wi1, wo1), (wi2, wo2))):
            in_rsem = sA_r if k == 0 else sAG_r

            for g in range(GROUPS):
                for q in range(g * GR, (g + 1) * GR):
                    @pl.when(q != me)
                    def _(q=q):
                        pltpu.make_async_remote_copy(
                            src_ref=x_ref,
                            dst_ref=xfull.at[rows(q)],
                            send_sem=sA_s.at[q],
                            recv_sem=in_rsem.at[q],
                            device_id=(q,),
                            device_id_type=pl.DeviceIdType.MESH,
                        ).wait_recv()

                gs = (pl.ds(g * GROWS, GROWS), slice(None))
                hg = jnp.dot(xfull[gs], win[...],
                             preferred_element_type=jnp.float32)
                h_ref[gs] = jnp.maximum(hg, 0.0).astype(jnp.bfloat16)
                pg = jnp.dot(h_ref[gs], wout[...],
                             preferred_element_type=jnp.float32)
                part_ref[gs] = pg.astype(jnp.bfloat16)

                @pl.when(g_me == g)
                def _():
                    rs_buf[rows(me)] = part_ref[rows(me)]

                for q in range(g * GR, (g + 1) * GR):
                    @pl.when(q != me)
                    def _(q=q):
                        pltpu.make_async_remote_copy(
                            src_ref=part_ref.at[rows(q)],
                            dst_ref=rs_buf.at[rows(me)],
                            send_sem=sRS_s.at[q],
                            recv_sem=sRS_r.at[me],
                            device_id=(q,),
                            device_id_type=pl.DeviceIdType.MESH,
                        ).start()

            def rs_wait_recv(d, c):
                p = peer(d)
                pltpu.make_async_remote_copy(
                    src_ref=part_ref.at[rows(me)],
                    dst_ref=rs_buf.at[rows(p)],
                    send_sem=sRS_s.at[p],
                    recv_sem=sRS_r.at[p],
                    device_id=(p,),
                    device_id_type=pl.DeviceIdType.MESH,
                ).wait_recv()
                return c
            lax.fori_loop(1, N_DEV, rs_wait_recv, 0)

            def rs_wait_send(d, c):
                p = peer(d)
                pltpu.make_async_remote_copy(
                    src_ref=part_ref.at[rows(p)],
                    dst_ref=rs_buf.at[rows(me)],
                    send_sem=sRS_s.at[p],
                    recv_sem=sRS_r.at[me],
                    device_id=(p,),
                    device_id_type=pl.DeviceIdType.MESH,
                ).wait_send()
                return c
            lax.fori_loop(1, N_DEV, rs_wait_send, 0)

            red = jnp.sum(
                rs_buf[...].astype(jnp.float32).reshape(N_DEV, BR, D),
                axis=0,
            )

            tgt = xfull if k < 2 else out_ref
            tgt[rows(me)] = red.astype(jnp.bfloat16)
            bcast_start(tgt.at[rows(me)], tgt.at[rows(me)], sAG_s, sAG_r)
            bcast_wait_send(tgt.at[rows(me)], tgt.at[rows(me)], sAG_s, sAG_r)

        def out_wait_recv(d, c):
            p = peer(d)
            pltpu.make_async_remote_copy(
                src_ref=out_ref.at[rows(me)],
                dst_ref=out_ref.at[rows(p)],
                send_sem=sAG_s.at[p],
                recv_sem=sAG_r.at[p],
                device_id=(p,),
                device_id_type=pl.DeviceIdType.MESH,
            ).wait_recv()
            return c
        lax.fori_loop(1, N_DEV, out_wait_recv, 0)

        bcast_wait_send(x_ref, xfull.at[rows(me)], sA_s, sA_r)

    vmem = pl.BlockSpec(memory_space=pltpu.VMEM)
    out = pl.pallas_call(
        body,
        out_shape=jax.ShapeDtypeStruct((B, D), jnp.bfloat16),
        in_specs=[vmem] * 7,
        out_specs=vmem,
        scratch_shapes=[
            pltpu.VMEM((B, D), jnp.bfloat16),
            pltpu.VMEM((B, H), jnp.bfloat16),
            pltpu.VMEM((B, D), jnp.bfloat16),
            pltpu.VMEM((B, D), jnp.bfloat16),
            pltpu.SemaphoreType.DMA((N_DEV,)),
            pltpu.SemaphoreType.DMA((N_DEV,)),
            pltpu.SemaphoreType.DMA((N_DEV,)),
            pltpu.SemaphoreType.DMA((N_DEV,)),
            pltpu.SemaphoreType.DMA((N_DEV,)),
            pltpu.SemaphoreType.DMA((N_DEV,)),
        ],
        compiler_params=pltpu.CompilerParams(
            vmem_limit_bytes=128 * 1024 * 1024,
            collective_id=0,
        ),
    )(xb, wi[0], wi[1], wi[2], wo[0], wo[1], wo[2])
    return out.astype(jnp.float32)


# device time: 426243 ns/iter; 1.1367x vs baseline; 1.1367x over previous
import jax
import jax.numpy as jnp
from jax import lax
from jax.experimental import pallas as pl
from jax.experimental.pallas import tpu as pltpu

N_DEV = 32
BR = 64
D = 1024
H = 2048
B = N_DEV * BR
NG = 8
GR = N_DEV // NG
GROWS = GR * BR


def kernel(x, Win0, Wout0, Win1, Wout1, Win2, Wout2):
    xb = x.astype(jnp.bfloat16)
    wi = [w.astype(jnp.bfloat16) for w in (Win0, Win1, Win2)]
    wo = [w.astype(jnp.bfloat16) for w in (Wout0, Wout1, Wout2)]

    def body(x_ref, wi0, wi1, wi2, wo0, wo1, wo2, out_ref,
             xfull, h_ref, part_ref, rs_buf,
             sA_s, sA_r, sRS_s, sRS_r, sAG_s, sAG_r):
        me = lax.axis_index("i")
        g_me = lax.div(me, GR)

        def peer(d):
            return lax.rem(me + d, N_DEV)

        rows = lambda i: (pl.ds(i * BR, BR), slice(None))

        barrier_sem = pltpu.get_barrier_semaphore()

        def bar(d, c):
            pl.semaphore_signal(
                barrier_sem, inc=1,
                device_id=(peer(d),),
                device_id_type=pl.DeviceIdType.MESH,
            )
            return c
        lax.fori_loop(1, N_DEV, bar, 0)
        pl.semaphore_wait(barrier_sem, N_DEV - 1)

        def bcast_start(src, dst, ssem, rsem, wait_send=False):
            def go(i, c):
                dg = lax.div(i, GR)
                t = lax.rem(i, GR)
                rg = lax.rem(g_me - dg + NG, NG)
                r = rg * GR + t
                @pl.when(r != me)
                def _():
                    rdma = pltpu.make_async_remote_copy(
                        src_ref=src,
                        dst_ref=dst,
                        send_sem=ssem.at[r],
                        recv_sem=rsem.at[me],
                        device_id=(r,),
                        device_id_type=pl.DeviceIdType.MESH,
                    )
                    if wait_send:
                        rdma.wait_send()
                    else:
                        rdma.start()
                return c
            lax.fori_loop(0, N_DEV, go, 0)

        xfull[rows(me)] = x_ref[...]
        bcast_start(x_ref, xfull.at[rows(me)], sA_s, sA_r)

        for k, (win, wout) in enumerate(((wi0, wo0), (wi1, wo1), (wi2, wo2))):
            in_rsem = sA_r if k == 0 else sAG_r

            def step(j, c, win=win, wout=wout, in_rsem=in_rsem):
                grp = lax.rem(g_me + j, NG)
                base = grp * GR
                for t in range(GR):
                    q = base + t
                    @pl.when(q != me)
                    def _(q=q):
                        pltpu.make_async_remote_copy(
                            src_ref=x_ref,
                            dst_ref=xfull.at[rows(q)],
                            send_sem=sA_s.at[q],
                            recv_sem=in_rsem.at[q],
                            device_id=(q,),
                            device_id_type=pl.DeviceIdType.MESH,
                        ).wait_recv()

                gs = (pl.ds(base * BR, GROWS), slice(None))
                hg = jnp.dot(xfull[gs], win[...],
                             preferred_element_type=jnp.float32)
                h_ref[gs] = jnp.maximum(hg, 0.0).astype(jnp.bfloat16)
                pg = jnp.dot(h_ref[gs], wout[...],
                             preferred_element_type=jnp.float32)
                part_ref[gs] = pg.astype(jnp.bfloat16)

                @pl.when(grp == g_me)
                def _():
                    rs_buf[rows(me)] = part_ref[rows(me)]

                for t in range(GR):
                    q = base + t
                    @pl.when(q != me)
                    def _(q=q):
                        pltpu.make_async_remote_copy(
                            src_ref=part_ref.at[rows(q)],
                            dst_ref=rs_buf.at[rows(me)],
                            send_sem=sRS_s.at[q],
                            recv_sem=sRS_r.at[me],
                            device_id=(q,),
                            device_id_type=pl.DeviceIdType.MESH,
                        ).start()
                return c
            lax.fori_loop(0, NG, step, 0)

            def rs_wait_recv(d, c):
                p = peer(d)
                pltpu.make_async_remote_copy(
                    src_ref=part_ref.at[rows(me)],
                    dst_ref=rs_buf.at[rows(p)],
                    send_sem=sRS_s.at[p],
                    recv_sem=sRS_r.at[p],
                    device_id=(p,),
                    device_id_type=pl.DeviceIdType.MESH,
                ).wait_recv()
                return c
            lax.fori_loop(1, N_DEV, rs_wait_recv, 0)

            def rs_wait_send(d, c):
                p = peer(d)
                pltpu.make_async_remote_copy(
                    src_ref=part_ref.at[rows(p)],
                    dst_ref=rs_buf.at[rows(me)],
                    send_sem=sRS_s.at[p],
                    recv_sem=sRS_r.at[me],
                    device_id=(p,),
                    device_id_type=pl.DeviceIdType.MESH,
                ).wait_send()
                return c
            lax.fori_loop(1, N_DEV, rs_wait_send, 0)

            red = jnp.sum(
                rs_buf[...].astype(jnp.float32).reshape(N_DEV, BR, D),
                axis=0,
            )

            tgt = xfull if k < 2 else out_ref
            tgt[rows(me)] = red.astype(jnp.bfloat16)
            bcast_start(tgt.at[rows(me)], tgt.at[rows(me)], sAG_s, sAG_r)
            bcast_start(tgt.at[rows(me)], tgt.at[rows(me)], sAG_s, sAG_r,
                        wait_send=True)

        def out_wait_recv(d, c):
            p = peer(d)
            pltpu.make_async_remote_copy(
                src_ref=out_ref.at[rows(me)],
                dst_ref=out_ref.at[rows(p)],
                send_sem=sAG_s.at[p],
                recv_sem=sAG_r.at[p],
                device_id=(p,),
                device_id_type=pl.DeviceIdType.MESH,
            ).wait_recv()
            return c
        lax.fori_loop(1, N_DEV, out_wait_recv, 0)

        bcast_start(x_ref, xfull.at[rows(me)], sA_s, sA_r, wait_send=True)

    vmem = pl.BlockSpec(memory_space=pltpu.VMEM)
    out = pl.pallas_call(
        body,
        out_shape=jax.ShapeDtypeStruct((B, D), jnp.bfloat16),
        in_specs=[vmem] * 7,
        out_specs=vmem,
        scratch_shapes=[
            pltpu.VMEM((B, D), jnp.bfloat16),
            pltpu.VMEM((B, H), jnp.bfloat16),
            pltpu.VMEM((B, D), jnp.bfloat16),
            pltpu.VMEM((B, D), jnp.bfloat16),
            pltpu.SemaphoreType.DMA((N_DEV,)),
            pltpu.SemaphoreType.DMA((N_DEV,)),
            pltpu.SemaphoreType.DMA((N_DEV,)),
            pltpu.SemaphoreType.DMA((N_DEV,)),
            pltpu.SemaphoreType.DMA((N_DEV,)),
            pltpu.SemaphoreType.DMA((N_DEV,)),
        ],
        compiler_params=pltpu.CompilerParams(
            vmem_limit_bytes=128 * 1024 * 1024,
            collective_id=0,
        ),
    )(xb, wi[0], wi[1], wi[2], wo[0], wo[1], wo[2])
    return out.astype(jnp.float32)
